# baseline (speedup 1.0000x reference)
import functools

import jax
import jax.numpy as jnp
from jax import lax
from jax.experimental import pallas as pl
from jax.experimental.pallas import tpu as pltpu
from jax.experimental.pallas import tpu_sc as plsc

W = H = C = 384
N = W * H
NW = 32
LANES = 16
CT = C // 128
NSLAB = N // 8
SLABS_PER_W = NSLAB // NW
S = 4
NBLK = SLABS_PER_W // S
ROWS = S * 8
NIN = 3
NOUT = 2

_mesh = plsc.VectorSubcoreMesh(core_axis_name="c", subcore_axis_name="s")


@functools.partial(
    pl.kernel,
    mesh=_mesh,
    compiler_params=pltpu.CompilerParams(needs_layout_passes=False),
    out_type=jax.ShapeDtypeStruct((NSLAB, CT, 8, 128), jnp.float32),
    scratch_types=(
        [pltpu.VMEM((S, CT, 8, 128), jnp.float32)] * NIN
        + [pltpu.VMEM((S, CT, 8, 128), jnp.int32)] * NIN
        + [pltpu.VMEM((S, CT, 8, 128), jnp.float32)] * NOUT
        + [pltpu.SemaphoreType.DMA] * (NIN + NOUT)
    ),
)
def _permute(img_hbm, perm_hbm, out_hbm,
             img_v0, img_v1, img_v2, perm_v0, perm_v1, perm_v2,
             out_v0, out_v1, sin0, sin1, sin2, sout0, sout1):
    img_v = (img_v0, img_v1, img_v2)
    perm_v = (perm_v0, perm_v1, perm_v2)
    out_v = (out_v0, out_v1)
    sin = (sin0, sin1, sin2)
    sout = (sout0, sout1)
    wid = lax.axis_index("s") * 2 + lax.axis_index("c")
    w_base = wid * SLABS_PER_W

    def start_in(blk, b):
        base = w_base + blk * S
        pltpu.async_copy(img_hbm.at[pl.ds(base, S)], img_v[b], sin[b])
        pltpu.async_copy(perm_hbm.at[pl.ds(base, S)], perm_v[b], sin[b])

    for b in range(NIN):
        start_in(b, b)

    def group_body(g, carry):
        for pos in range(6):
            ib = pos % NIN
            ob = pos % NOUT
            blk = g * 6 + pos
            base = w_base + blk * S
            pltpu.make_async_copy(img_hbm.at[pl.ds(base, S)],
                                  img_v[ib], sin[ib]).wait()
            pltpu.make_async_copy(perm_hbm.at[pl.ds(base, S)],
                                  perm_v[ib], sin[ib]).wait()
            @pl.when(blk >= NOUT)
            def _():
                pltpu.make_async_copy(out_v[ob],
                                      out_hbm.at[pl.ds(base, S)],
                                      sout[ob]).wait()

            @plsc.parallel_loop(0, ROWS, unroll=4)
            def pix_body(pix):
                slab = lax.shift_right_logical(pix, 3)
                r = lax.bitwise_and(pix, 7)
                slab_splat = jnp.full((LANES,), slab, jnp.int32)
                r_splat = jnp.full((LANES,), r, jnp.int32)
                for tc in range(CT):
                    for k in range(128 // LANES):
                        p = perm_v[ib][slab, tc, r, pl.ds(k * LANES, LANES)]
                        tc_src = lax.shift_right_logical(p, 7)
                        l_src = lax.bitwise_and(p, 127)
                        out_v[ob][slab, tc, r, pl.ds(k * LANES, LANES)] = (
                            plsc.load_gather(
                                img_v[ib],
                                [slab_splat, tc_src, r_splat, l_src]))

            pltpu.async_copy(out_v[ob], out_hbm.at[pl.ds(base, S)], sout[ob])
            @pl.when(blk + NIN < NBLK)
            def _():
                start_in(blk + NIN, ib)

        return carry

    lax.fori_loop(0, NBLK // 6, group_body, 0)

    for blk in range(NBLK - NOUT, NBLK):
        base = w_base + blk * S
        pltpu.make_async_copy(out_v[blk % NOUT], out_hbm.at[pl.ds(base, S)],
                              sout[blk % NOUT]).wait()


def _to_tiled(x):
    return (x.reshape(W, H // 8, 8, CT, 128)
             .transpose(0, 1, 3, 2, 4)
             .reshape(NSLAB, CT, 8, 128))


def kernel(image, perm):
    out = _permute(_to_tiled(image), _to_tiled(perm))
    return (out.reshape(W, H // 8, CT, 8, 128)
               .transpose(0, 1, 3, 2, 4)
               .reshape(W, H, C))


# speedup vs baseline: 1.1025x; 1.1025x over previous
import functools

import jax
import jax.numpy as jnp
from jax import lax
from jax.experimental import pallas as pl
from jax.experimental.pallas import tpu as pltpu
from jax.experimental.pallas import tpu_sc as plsc

W = H = C = 384
N = W * H
NW = 32
LANES = 16
CT = C // 128
NSLAB = N // 8
SLABS_PER_W = NSLAB // NW
S = 6
NBLK = SLABS_PER_W // S
ROWS = S * 8

_mesh = plsc.VectorSubcoreMesh(core_axis_name="c", subcore_axis_name="s")


@functools.partial(
    pl.kernel,
    mesh=_mesh,
    compiler_params=pltpu.CompilerParams(needs_layout_passes=False),
    out_type=jax.ShapeDtypeStruct((NSLAB, CT, 8, 128), jnp.float32),
    scratch_types=[
        pltpu.VMEM((S, CT, 8, 128), jnp.float32),
        pltpu.VMEM((S, CT, 8, 128), jnp.float32),
        pltpu.VMEM((S, CT, 8, 128), jnp.int32),
        pltpu.VMEM((S, CT, 8, 128), jnp.int32),
        pltpu.VMEM((S, CT, 8, 128), jnp.float32),
        pltpu.VMEM((S, CT, 8, 128), jnp.float32),
        pltpu.SemaphoreType.DMA,
        pltpu.SemaphoreType.DMA,
        pltpu.SemaphoreType.DMA,
        pltpu.SemaphoreType.DMA,
    ],
)
def _permute(img_hbm, perm_hbm, out_hbm, img_v0, img_v1, perm_v0, perm_v1,
             out_v0, out_v1, sin0, sin1, sout0, sout1):
    img_v = (img_v0, img_v1)
    perm_v = (perm_v0, perm_v1)
    out_v = (out_v0, out_v1)
    sin = (sin0, sin1)
    sout = (sout0, sout1)
    wid = lax.axis_index("s") * 2 + lax.axis_index("c")
    w_base = wid * SLABS_PER_W

    def start_in(blk, b):
        base = w_base + blk * S
        pltpu.async_copy(img_hbm.at[pl.ds(base, S)], img_v[b], sin[b])
        pltpu.async_copy(perm_hbm.at[pl.ds(base, S)], perm_v[b], sin[b])

    for b in range(2):
        start_in(b, b)

    def group_body(g, carry):
        for b in range(2):
            blk = g * 2 + b
            base = w_base + blk * S
            pltpu.make_async_copy(img_hbm.at[pl.ds(base, S)],
                                  img_v[b], sin[b]).wait()
            pltpu.make_async_copy(perm_hbm.at[pl.ds(base, S)],
                                  perm_v[b], sin[b]).wait()
            @pl.when(g > 0)
            def _():
                pltpu.make_async_copy(out_v[b],
                                      out_hbm.at[pl.ds(base, S)],
                                      sout[b]).wait()

            @plsc.parallel_loop(0, ROWS, unroll=4)
            def pix_body(pix):
                slab = lax.shift_right_logical(pix, 2 + 1)
                r = lax.bitwise_and(pix, 7)
                slab_splat = jnp.full((LANES,), slab, jnp.int32)
                r_splat = jnp.full((LANES,), r, jnp.int32)
                for tc in range(CT):
                    for k in range(128 // LANES):
                        p = perm_v[b][slab, tc, r, pl.ds(k * LANES, LANES)]
                        tc_src = lax.shift_right_logical(p, 7)
                        l_src = lax.bitwise_and(p, 127)
                        out_v[b][slab, tc, r, pl.ds(k * LANES, LANES)] = (
                            plsc.load_gather(
                                img_v[b], [slab_splat, tc_src, r_splat, l_src]))

            @pl.when(blk + 2 < NBLK)
            def _():
                start_in(blk + 2, b)

            pltpu.async_copy(out_v[b], out_hbm.at[pl.ds(base, S)], sout[b])

        return carry

    lax.fori_loop(0, NBLK // 2, group_body, 0)

    for b in range(2):
        blk = NBLK - 2 + b
        base = w_base + blk * S
        pltpu.make_async_copy(out_v[b], out_hbm.at[pl.ds(base, S)],
                              sout[b]).wait()


def _to_tiled(x):
    return (x.reshape(W, H // 8, 8, CT, 128)
             .transpose(0, 1, 3, 2, 4)
             .reshape(NSLAB, CT, 8, 128))


def kernel(image, perm):
    out = _permute(_to_tiled(image), _to_tiled(perm))
    return (out.reshape(W, H // 8, CT, 8, 128)
               .transpose(0, 1, 3, 2, 4)
               .reshape(W, H, C))
